# baseline (speedup 1.0000x reference)
import functools

import jax
import jax.numpy as jnp
from jax.experimental import pallas as pl
from jax.experimental.pallas import tpu as pltpu
from jax.experimental.pallas import tpu_sc as plsc

N = 10000
T = 20000
L = 128
NODE_TYPE_SIZE = 9
OBSTACLE = 1
RADIUS = 0.03
STEPS = 2
WORLD_EDGE_CAP = 131072

NPAD = 10240
EM = 6 * T
EM_PAD = 120320
EW = WORLD_EDGE_CAP
ROW_BLK = 256
MLP_BLK = 512

NG = NPAD // 16
NSUB = 32
GPS = NG // NSUB
GCAP = 4096


def _radius_packed_kernel(wp_ref, wpt_ref, x2_ref, colmask_ref, packed_ref):
    i = pl.program_id(0)
    wp = wp_ref[...]
    row_sq = jnp.sum(wp * wp, axis=1, keepdims=True)
    cross = jax.lax.dot_general(
        wp, wpt_ref[...], (((1,), (0,)), ((), ())),
        preferred_element_type=jnp.float32)
    d2 = row_sq + x2_ref[...] - 2.0 * cross
    dist = jnp.sqrt(jnp.maximum(d2, 0.0))
    rows = i * ROW_BLK + jax.lax.broadcasted_iota(jnp.int32, (ROW_BLK, NPAD), 0)
    cols = jax.lax.broadcasted_iota(jnp.int32, (ROW_BLK, NPAD), 1)
    conn = (dist < RADIUS) & (rows != cols) & (rows < N) & (cols < N)
    conn = conn & colmask_ref[...]
    rr = jax.lax.broadcasted_iota(jnp.int32, (16, ROW_BLK), 1)
    tt = jax.lax.broadcasted_iota(jnp.int32, (16, ROW_BLK), 0)
    a = jnp.where(rr // 16 == tt,
                  jax.lax.shift_left(jnp.int32(1), rr % 16), 0
                  ).astype(jnp.float32)
    packed_f = jax.lax.dot_general(
        a, conn.astype(jnp.float32), (((1,), (0,)), ((), ())),
        preferred_element_type=jnp.float32)
    packed_ref[...] = packed_f.astype(jnp.int32)


def _radius_packed(world_pos, colmask):
    wp_pad = jnp.zeros((NPAD, 8), jnp.float32)
    wp_pad = wp_pad.at[:, 0].set(1e6)
    wp_pad = wp_pad.at[:N, :3].set(world_pos)
    wp_pad = wp_pad.at[:N, 3:].set(0.0)
    x2 = jnp.sum(wp_pad * wp_pad, axis=1)[None, :]
    packed = pl.pallas_call(
        _radius_packed_kernel,
        grid=(NPAD // ROW_BLK,),
        in_specs=[
            pl.BlockSpec((ROW_BLK, 8), lambda i: (i, 0)),
            pl.BlockSpec((8, NPAD), lambda i: (0, 0)),
            pl.BlockSpec((1, NPAD), lambda i: (0, 0)),
            pl.BlockSpec((1, NPAD), lambda i: (0, 0)),
        ],
        out_specs=pl.BlockSpec((16, NPAD), lambda i: (i, 0)),
        out_shape=jax.ShapeDtypeStruct((NG, NPAD), jnp.int32),
    )(wp_pad, wp_pad.T, x2, colmask.reshape(1, NPAD))
    return packed


def _sget(vec, chunk, lane):
    v = vec[pl.ds(chunk * 16, 16)]
    return jnp.sum(jnp.where(jax.lax.iota(jnp.int32, 16) == lane, v, 0))


def _expand_body(packed_hbm, meta_hbm, ws_hbm, wr_hbm,
                 meta_v, row_v, stage_s, stage_r):
    c = jax.lax.axis_index("c")
    s = jax.lax.axis_index("s")
    wid = s * 2 + c
    pltpu.sync_copy(meta_hbm.at[wid], meta_v)
    for g in range(GPS):
        off = _sget(meta_v, g // 16, g % 16)
        cnt = _sget(meta_v, 2 + g // 16, g % 16)
        grp = wid * GPS + g

        @pl.when(cnt >= 0)
        def _process():
            pltpu.sync_copy(packed_hbm.at[grp], row_v)

            def scan_body(k, fill):
                w = row_v[pl.ds(k * 16, 16)]
                nz = jnp.sum(jnp.where(w != 0, 1, 0))

                def emit(f):
                    for b in range(16):
                        maskb = ((w >> b) & 1) == 1
                        cb = jnp.sum(jnp.where(maskb, 1, 0))

                        def do(f2):
                            f2c = jnp.minimum(f2, GCAP - 16)
                            cols = k * 16 + jax.lax.iota(jnp.int32, 16)
                            rowv = jnp.full((16,), 0, jnp.int32) + grp * 16 + b
                            plsc.store_compressed(
                                stage_s.at[pl.ds(f2c, 16)], rowv, mask=maskb)
                            plsc.store_compressed(
                                stage_r.at[pl.ds(f2c, 16)], cols, mask=maskb)
                            return f2 + cb

                        f = jax.lax.cond(cb > 0, do, lambda f2: f2, f)
                    return f

                return jax.lax.cond(nz > 0, emit, lambda f: f, fill)

            fill = jax.lax.fori_loop(0, NPAD // 16, scan_body, 0)
            fillc = jnp.minimum(fill, GCAP - 16)
            sent = jnp.full((16,), 0, jnp.int32) + N
            stage_s[pl.ds(fillc, 16)] = sent
            stage_r[pl.ds(fillc, 16)] = sent

            def drain(ci, _):
                src = pl.multiple_of(ci * 16, 16)
                dst = pl.multiple_of(off + ci * 16, 16)
                pltpu.sync_copy(stage_s.at[pl.ds(src, 16)],
                                ws_hbm.at[pl.ds(dst, 16)])
                pltpu.sync_copy(stage_r.at[pl.ds(src, 16)],
                                wr_hbm.at[pl.ds(dst, 16)])
                return 0

            jax.lax.fori_loop(0, fillc // 16 + 1, drain, 0)


def _expand_edges(packed, meta):
    return pl.kernel(
        _expand_body,
        out_type=[
            jax.ShapeDtypeStruct((EW,), jnp.int32),
            jax.ShapeDtypeStruct((EW,), jnp.int32),
        ],
        mesh=plsc.VectorSubcoreMesh(core_axis_name="c", subcore_axis_name="s"),
        compiler_params=pltpu.CompilerParams(needs_layout_passes=False),
        scratch_types=[
            pltpu.VMEM((64,), jnp.int32),
            pltpu.VMEM((NPAD,), jnp.int32),
            pltpu.VMEM((GCAP + 16,), jnp.int32),
            pltpu.VMEM((GCAP + 16,), jnp.int32),
        ],
    )(packed, meta)


GCH_M = 376
NCH_M = 10
GCH_W = 512
NCH_W = 8


def _gather4_body(ps_me, pr_me, ps_we, pr_we, sid, rid, wsid, wrid,
                  gs_me, gr_me, gs_we, gr_we,
                  idx_m, idx_w, rows_m, rows_w, sem):
    c = jax.lax.axis_index("c")
    s = jax.lax.axis_index("s")
    wid = s * 2 + c

    def mesh_gather(table, ids, out):
        base = wid * (NCH_M * GCH_M)
        for k in range(NCH_M):
            start = base + k * GCH_M
            pltpu.sync_copy(ids.at[pl.ds(start, GCH_M)], idx_m)
            pltpu.async_copy(table.at[idx_m], rows_m, sem).wait()
            pltpu.sync_copy(rows_m, out.at[pl.ds(start, GCH_M)])

    def world_gather(table, ids, out):
        base = wid * (NCH_W * GCH_W)
        for k in range(NCH_W):
            start = base + k * GCH_W
            pltpu.sync_copy(ids.at[pl.ds(start, GCH_W)], idx_w)
            pltpu.async_copy(table.at[idx_w], rows_w, sem).wait()
            pltpu.sync_copy(rows_w, out.at[pl.ds(start, GCH_W)])

    mesh_gather(ps_me, sid, gs_me)
    mesh_gather(pr_me, rid, gr_me)
    world_gather(ps_we, wsid, gs_we)
    world_gather(pr_we, wrid, gr_we)


def _gather4(ps_me, pr_me, ps_we, pr_we, sid, rid, wsid, wrid):
    return pl.kernel(
        _gather4_body,
        out_type=[
            jax.ShapeDtypeStruct((EM_PAD, L), jnp.float32),
            jax.ShapeDtypeStruct((EM_PAD, L), jnp.float32),
            jax.ShapeDtypeStruct((EW, L), jnp.float32),
            jax.ShapeDtypeStruct((EW, L), jnp.float32),
        ],
        mesh=plsc.VectorSubcoreMesh(core_axis_name="c", subcore_axis_name="s"),
        compiler_params=pltpu.CompilerParams(needs_layout_passes=False),
        scratch_types=[
            pltpu.VMEM((GCH_M,), jnp.int32),
            pltpu.VMEM((GCH_W,), jnp.int32),
            pltpu.VMEM((GCH_M, L), jnp.float32),
            pltpu.VMEM((GCH_W, L), jnp.float32),
            pltpu.SemaphoreType.DMA,
        ],
    )(ps_me, pr_me, ps_we, pr_we, sid, rid, wsid, wrid)


def _world_edge_lists(world_pos, node_type, uniq, s0, r0):
    obstacle = node_type[:, 0] == OBSTACLE
    colmask = jnp.pad(~obstacle, (0, NPAD - N))
    packed = _radius_packed(world_pos, colmask)
    sd = jnp.concatenate([s0, r0])
    rd = jnp.concatenate([r0, s0])
    dup_self = jnp.concatenate(
        [jnp.zeros_like(s0, jnp.bool_), s0 == r0])
    ok = (sd < N) & (rd < N) & ~dup_self
    flat_idx = jnp.where(ok, (sd // 16) * NPAD + rd, NG * NPAD)
    clear = jnp.zeros((NG * NPAD,), jnp.int32).at[flat_idx].add(
        jax.lax.shift_left(jnp.int32(1), sd % 16), mode='drop')
    packed = packed & ~clear.reshape(NG, NPAD)
    counts = jnp.sum(jax.lax.population_count(packed), axis=1)
    ccl = jnp.minimum(counts, GCAP - 16)
    slots = 16 * (ccl // 16 + 1)
    off = jnp.concatenate([jnp.zeros((1,), jnp.int32),
                           jnp.cumsum(slots)[:-1].astype(jnp.int32)])
    okg = (off + slots) <= EW
    cntm = jnp.where(okg, ccl, -1).astype(jnp.int32)
    offm = jnp.where(okg, off, 0).astype(jnp.int32)
    meta = jnp.zeros((NSUB, 64), jnp.int32)
    meta = meta.at[:, 0:GPS].set(offm.reshape(NSUB, GPS))
    meta = meta.at[:, 32:32 + GPS].set(cntm.reshape(NSUB, GPS))
    ws, wr = _expand_edges(packed, meta)
    total = jnp.sum(jnp.where(okg, slots, 0))
    valid = jnp.arange(EW) < total
    ws = jnp.where(valid, ws, N)
    wr = jnp.where(valid, wr, N)
    return ws, wr


def _fused_mlp_body(nx, weighted, ln, res_idx, *refs):
    xs = refs[:nx]
    nw = sum(weighted)
    w1s = refs[nx:nx + nw]
    b1_ref, w2_ref, b2_ref = refs[nx + nw:nx + nw + 3]
    out_ref = refs[-1]
    h = b1_ref[...]
    wi = 0
    for i in range(nx):
        x = xs[i][...]
        if weighted[i]:
            h = h + jax.lax.dot_general(
                x, w1s[wi][...], (((1,), (0,)), ((), ())),
                preferred_element_type=jnp.float32)
            wi += 1
        else:
            h = h + x
    h = jnp.maximum(h, 0.0)
    o = jax.lax.dot_general(
        h, w2_ref[...], (((1,), (0,)), ((), ())),
        preferred_element_type=jnp.float32) + b2_ref[...]
    if ln:
        m = jnp.mean(o, axis=-1, keepdims=True)
        d = o - m
        v = jnp.mean(d * d, axis=-1, keepdims=True)
        o = d * jax.lax.rsqrt(v + 1e-5)
    if res_idx is not None:
        o = o + xs[res_idx][...]
    out_ref[...] = o


def _fused_mlp(xs, w1s, b1, w2, b2, ln=True, res_idx=None, dout=L):
    M = xs[0].shape[0]
    weighted = tuple(w is not None for w in w1s)
    body = functools.partial(_fused_mlp_body, len(xs), weighted, ln, res_idx)
    in_specs = []
    args = []
    for x in xs:
        d = x.shape[1]
        in_specs.append(pl.BlockSpec((MLP_BLK, d), lambda i: (i, 0)))
        args.append(x)
    for w in w1s:
        if w is not None:
            in_specs.append(pl.BlockSpec(w.shape, lambda i: (0, 0)))
            args.append(w)
    for c in (b1.reshape(1, -1), w2, b2.reshape(1, -1)):
        in_specs.append(pl.BlockSpec(c.shape, lambda i: (0, 0)))
        args.append(c)
    return pl.pallas_call(
        body,
        grid=(M // MLP_BLK,),
        in_specs=in_specs,
        out_specs=pl.BlockSpec((MLP_BLK, dout), lambda i: (i, 0)),
        out_shape=jax.ShapeDtypeStruct((M, dout), jnp.float32),
    )(*args)


def _matmul_kernel(x_ref, w_ref, out_ref):
    out_ref[...] = jax.lax.dot_general(
        x_ref[...], w_ref[...], (((1,), (0,)), ((), ())),
        preferred_element_type=jnp.float32)


def _matmul(x, w):
    M = x.shape[0]
    return pl.pallas_call(
        _matmul_kernel,
        grid=(M // MLP_BLK,),
        in_specs=[
            pl.BlockSpec((MLP_BLK, x.shape[1]), lambda i: (i, 0)),
            pl.BlockSpec(w.shape, lambda i: (0, 0)),
        ],
        out_specs=pl.BlockSpec((MLP_BLK, w.shape[1]), lambda i: (i, 0)),
        out_shape=jax.ShapeDtypeStruct((M, w.shape[1]), jnp.float32),
    )(x, w)


def _pad_rows(x, M):
    return jnp.pad(x, ((0, M - x.shape[0]), (0, 0)))


def _pad_cols(x, D):
    return jnp.pad(x, ((0, 0), (0, D - x.shape[1])))


def _safe_norm(x):
    return jnp.sqrt(jnp.sum(x * x, axis=-1, keepdims=True) + 1e-12)


def _mesh_edge_lists(cells):
    e = jnp.concatenate([cells[:, 0:2], cells[:, 1:3],
                         jnp.stack([cells[:, 2], cells[:, 0]], axis=1)], axis=0)
    lo = jnp.minimum(e[:, 0], e[:, 1])
    hi = jnp.maximum(e[:, 0], e[:, 1])
    uniq = jnp.unique(lo * N + hi, size=e.shape[0], fill_value=N * N)
    valid = uniq < N * N
    s0 = jnp.where(valid, uniq // N, N).astype(jnp.int32)
    r0 = jnp.where(valid, uniq % N, N).astype(jnp.int32)
    senders = jnp.concatenate([s0, r0])
    receivers = jnp.concatenate([r0, s0])
    return senders, receivers, uniq, s0, r0


def kernel(world_pos, prev_world_pos, mesh_pos, node_type, cells, params):
    p = params
    senders, receivers, uniq, s0, r0 = _mesh_edge_lists(cells)
    ws, wr = _world_edge_lists(world_pos, node_type, uniq, s0, r0)

    velocity = world_pos - prev_world_pos
    one_hot = jax.nn.one_hot(node_type[:, 0], NODE_TYPE_SIZE, dtype=jnp.float32)
    node_feats = _pad_rows(_pad_cols(
        jnp.concatenate([velocity, one_hot], axis=-1), 16), NPAD)
    node_lat = _fused_mlp(
        [node_feats], [_pad_rows(p['node_enc_w1'], 16)],
        p['node_enc_b1'], p['node_enc_w2'], p['node_enc_b2'])

    relw = world_pos[wr] - world_pos[ws]
    world_feats = _pad_cols(
        jnp.concatenate([relw, _safe_norm(relw)], axis=-1), 8)
    world_lat = _fused_mlp(
        [world_feats], [_pad_rows(p['world_enc_w1'], 8)],
        p['world_enc_b1'], p['world_enc_w2'], p['world_enc_b2'])

    relwm = world_pos[senders] - world_pos[receivers]
    relm = mesh_pos[senders] - mesh_pos[receivers]
    mesh_feats = _pad_rows(_pad_cols(jnp.concatenate(
        [relwm, _safe_norm(relwm), relm, _safe_norm(relm)], axis=-1), 8), EM_PAD)
    mesh_lat = _fused_mlp(
        [mesh_feats], [_pad_rows(p['mesh_enc_w1'], 8)],
        p['mesh_enc_b1'], p['mesh_enc_w2'], p['mesh_enc_b2'])

    me_w1 = p['me_w1']
    we_w1 = p['we_w1']
    nd_w1 = p['nd_w1']
    pcat_w = jnp.concatenate(
        [me_w1[:L], me_w1[L:2 * L], we_w1[:L], we_w1[L:2 * L]], axis=1)

    sid_cl = jnp.pad(jnp.minimum(senders, N - 1), (0, EM_PAD - EM))
    rid_cl = jnp.pad(jnp.minimum(receivers, N - 1), (0, EM_PAD - EM))
    wsid_cl = jnp.minimum(ws, N - 1)
    wrid_cl = jnp.minimum(wr, N - 1)

    for _ in range(STEPS):
        pcat = _matmul(node_lat, pcat_w)[:N]
        gs_me, gr_me, gs_we, gr_we = _gather4(
            pcat[:, 0:L], pcat[:, L:2 * L],
            pcat[:, 2 * L:3 * L], pcat[:, 3 * L:4 * L],
            sid_cl, rid_cl, wsid_cl, wrid_cl)

        mesh_lat = _fused_mlp(
            [gs_me, gr_me, mesh_lat], [None, None, me_w1[2 * L:]],
            p['me_b1'], p['me_w2'], p['me_b2'], res_idx=2)
        world_lat = _fused_mlp(
            [gs_we, gr_we, world_lat], [None, None, we_w1[2 * L:]],
            p['we_b1'], p['we_w2'], p['we_b2'], res_idx=2)

        agg_m = jax.ops.segment_sum(
            mesh_lat, jnp.pad(receivers, (0, EM_PAD - EM), constant_values=N),
            num_segments=N)
        agg_w = jax.ops.segment_sum(world_lat, wr, num_segments=N)
        node_lat = _fused_mlp(
            [node_lat, _pad_rows(agg_m, NPAD), _pad_rows(agg_w, NPAD)],
            [nd_w1[:L], nd_w1[L:2 * L], nd_w1[2 * L:]],
            p['nd_b1'], p['nd_w2'], p['nd_b2'], res_idx=0)

    out = _fused_mlp(
        [node_lat], [p['dec_w1']],
        p['dec_b1'], _pad_cols(p['dec_w2'], L),
        jnp.pad(p['dec_b2'], (0, L - 3)), ln=False)
    return out[:N, :3]


# speedup vs baseline: 1.8576x; 1.8576x over previous
import functools

import jax
import jax.numpy as jnp
from jax.experimental import pallas as pl
from jax.experimental.pallas import tpu as pltpu
from jax.experimental.pallas import tpu_sc as plsc

N = 10000
T = 20000
L = 128
NODE_TYPE_SIZE = 9
OBSTACLE = 1
RADIUS = 0.03
STEPS = 2
WORLD_EDGE_CAP = 131072

NPAD = 10240
EM = 6 * T
EM_PAD = 131072
EW = WORLD_EDGE_CAP
ROW_BLK = 256
MLP_BLK = 512

NG = NPAD // 16
NSUB = 32
GPS = NG // NSUB
GCAP = 4096


def _radius_packed_kernel(wp_ref, wpt_ref, x2_ref, colmask_ref, packed_ref):
    i = pl.program_id(0)
    wp = wp_ref[...]
    row_sq = jnp.sum(wp * wp, axis=1, keepdims=True)
    cross = jax.lax.dot_general(
        wp, wpt_ref[...], (((1,), (0,)), ((), ())),
        preferred_element_type=jnp.float32)
    d2 = row_sq + x2_ref[...] - 2.0 * cross
    dist = jnp.sqrt(jnp.maximum(d2, 0.0))
    rows = i * ROW_BLK + jax.lax.broadcasted_iota(jnp.int32, (ROW_BLK, NPAD), 0)
    cols = jax.lax.broadcasted_iota(jnp.int32, (ROW_BLK, NPAD), 1)
    conn = (dist < RADIUS) & (rows != cols) & (rows < N) & (cols < N)
    conn = conn & colmask_ref[...]
    rr = jax.lax.broadcasted_iota(jnp.int32, (16, ROW_BLK), 1)
    tt = jax.lax.broadcasted_iota(jnp.int32, (16, ROW_BLK), 0)
    a = jnp.where(rr // 16 == tt,
                  jax.lax.shift_left(jnp.int32(1), rr % 16), 0
                  ).astype(jnp.float32)
    packed_f = jax.lax.dot_general(
        a, conn.astype(jnp.float32), (((1,), (0,)), ((), ())),
        preferred_element_type=jnp.float32)
    packed_ref[...] = packed_f.astype(jnp.int32)


def _radius_packed(world_pos, colmask):
    wp_pad = jnp.zeros((NPAD, 8), jnp.float32)
    wp_pad = wp_pad.at[:, 0].set(1e6)
    wp_pad = wp_pad.at[:N, :3].set(world_pos)
    wp_pad = wp_pad.at[:N, 3:].set(0.0)
    x2 = jnp.sum(wp_pad * wp_pad, axis=1)[None, :]
    packed = pl.pallas_call(
        _radius_packed_kernel,
        grid=(NPAD // ROW_BLK,),
        in_specs=[
            pl.BlockSpec((ROW_BLK, 8), lambda i: (i, 0)),
            pl.BlockSpec((8, NPAD), lambda i: (0, 0)),
            pl.BlockSpec((1, NPAD), lambda i: (0, 0)),
            pl.BlockSpec((1, NPAD), lambda i: (0, 0)),
        ],
        out_specs=pl.BlockSpec((16, NPAD), lambda i: (i, 0)),
        out_shape=jax.ShapeDtypeStruct((NG, NPAD), jnp.int32),
    )(wp_pad, wp_pad.T, x2, colmask.reshape(1, NPAD))
    return packed


def _sget(vec, chunk, lane):
    v = vec[pl.ds(chunk * 16, 16)]
    return jnp.sum(jnp.where(jax.lax.iota(jnp.int32, 16) == lane, v, 0))


def _expand_body(packed_hbm, meta_hbm, ws_hbm, wr_hbm,
                 meta_v, row_v, stage_s, stage_r):
    c = jax.lax.axis_index("c")
    s = jax.lax.axis_index("s")
    wid = s * 2 + c
    pltpu.sync_copy(meta_hbm.at[wid], meta_v)
    for g in range(GPS):
        off = _sget(meta_v, g // 16, g % 16)
        cnt = _sget(meta_v, 2 + g // 16, g % 16)
        grp = wid * GPS + g

        @pl.when(cnt >= 0)
        def _process():
            pltpu.sync_copy(packed_hbm.at[grp], row_v)

            def scan_body(k, fill):
                w = row_v[pl.ds(k * 16, 16)]
                nz = jnp.sum(jnp.where(w != 0, 1, 0))

                def emit(f):
                    for b in range(16):
                        maskb = ((w >> b) & 1) == 1
                        cb = jnp.sum(jnp.where(maskb, 1, 0))

                        def do(f2):
                            f2c = jnp.minimum(f2, GCAP - 16)
                            cols = k * 16 + jax.lax.iota(jnp.int32, 16)
                            rowv = jnp.full((16,), 0, jnp.int32) + grp * 16 + b
                            plsc.store_compressed(
                                stage_s.at[pl.ds(f2c, 16)], rowv, mask=maskb)
                            plsc.store_compressed(
                                stage_r.at[pl.ds(f2c, 16)], cols, mask=maskb)
                            return f2 + cb

                        f = jax.lax.cond(cb > 0, do, lambda f2: f2, f)
                    return f

                return jax.lax.cond(nz > 0, emit, lambda f: f, fill)

            fill = jax.lax.fori_loop(0, NPAD // 16, scan_body, 0)
            fillc = jnp.minimum(fill, GCAP - 16)
            sent = jnp.full((16,), 0, jnp.int32) + N
            stage_s[pl.ds(fillc, 16)] = sent
            stage_r[pl.ds(fillc, 16)] = sent

            def drain(ci, _):
                src = pl.multiple_of(ci * 16, 16)
                dst = pl.multiple_of(off + ci * 16, 16)
                pltpu.sync_copy(stage_s.at[pl.ds(src, 16)],
                                ws_hbm.at[pl.ds(dst, 16)])
                pltpu.sync_copy(stage_r.at[pl.ds(src, 16)],
                                wr_hbm.at[pl.ds(dst, 16)])
                return 0

            jax.lax.fori_loop(0, fillc // 16 + 1, drain, 0)


def _expand_edges(packed, meta):
    return pl.kernel(
        _expand_body,
        out_type=[
            jax.ShapeDtypeStruct((EW,), jnp.int32),
            jax.ShapeDtypeStruct((EW,), jnp.int32),
        ],
        mesh=plsc.VectorSubcoreMesh(core_axis_name="c", subcore_axis_name="s"),
        compiler_params=pltpu.CompilerParams(needs_layout_passes=False),
        scratch_types=[
            pltpu.VMEM((64,), jnp.int32),
            pltpu.VMEM((NPAD,), jnp.int32),
            pltpu.VMEM((GCAP + 16,), jnp.int32),
            pltpu.VMEM((GCAP + 16,), jnp.int32),
        ],
    )(packed, meta)


GCH_M = 512
NCH_M = 8
GCH_W = 512
NCH_W = 8


def _gather4_body(ps_me, pr_me, ps_we, pr_we, sid, rid, wsid, wrid,
                  gs_me, gr_me, gs_we, gr_we,
                  idx_m, idx_w, rows_m, rows_w, sem):
    c = jax.lax.axis_index("c")
    s = jax.lax.axis_index("s")
    wid = s * 2 + c

    def mesh_gather(table, ids, out):
        base = wid * (NCH_M * GCH_M)
        for k in range(NCH_M):
            start = base + k * GCH_M
            pltpu.sync_copy(ids.at[pl.ds(start, GCH_M)], idx_m)
            pltpu.async_copy(table.at[idx_m], rows_m, sem).wait()
            pltpu.sync_copy(rows_m, out.at[pl.ds(start, GCH_M)])

    def world_gather(table, ids, out):
        base = wid * (NCH_W * GCH_W)
        for k in range(NCH_W):
            start = base + k * GCH_W
            pltpu.sync_copy(ids.at[pl.ds(start, GCH_W)], idx_w)
            pltpu.async_copy(table.at[idx_w], rows_w, sem).wait()
            pltpu.sync_copy(rows_w, out.at[pl.ds(start, GCH_W)])

    mesh_gather(ps_me, sid, gs_me)
    mesh_gather(pr_me, rid, gr_me)
    world_gather(ps_we, wsid, gs_we)
    world_gather(pr_we, wrid, gr_we)


def _gather4(ps_me, pr_me, ps_we, pr_we, sid, rid, wsid, wrid):
    return pl.kernel(
        _gather4_body,
        out_type=[
            jax.ShapeDtypeStruct((EM_PAD, L), jnp.float32),
            jax.ShapeDtypeStruct((EM_PAD, L), jnp.float32),
            jax.ShapeDtypeStruct((EW, L), jnp.float32),
            jax.ShapeDtypeStruct((EW, L), jnp.float32),
        ],
        mesh=plsc.VectorSubcoreMesh(core_axis_name="c", subcore_axis_name="s"),
        compiler_params=pltpu.CompilerParams(needs_layout_passes=False),
        scratch_types=[
            pltpu.VMEM((GCH_M,), jnp.int32),
            pltpu.VMEM((GCH_W,), jnp.int32),
            pltpu.VMEM((GCH_M, L), jnp.float32),
            pltpu.VMEM((GCH_W, L), jnp.float32),
            pltpu.SemaphoreType.DMA,
        ],
    )(ps_me, pr_me, ps_we, pr_we, sid, rid, wsid, wrid)


NAGG = 10240
SCH = 128
NCH_S = EW // 16 // SCH
NZB = NAGG // 16 // SCH


def _scatter2_body(mlat, wlat, rid, wrid, zeros_h, agg_m, agg_w,
                   idx_v, rows_v, acc_sh):
    c = jax.lax.axis_index("c")
    s = jax.lax.axis_index("s")
    for z in range(NZB):
        pltpu.sync_copy(zeros_h, acc_sh.at[pl.ds(s * (NZB * SCH) + z * SCH, SCH)])
    plsc.subcore_barrier()
    base = s * (NCH_S * SCH)

    @pl.when(c == 0)
    def _mesh():
        for k in range(NCH_S):
            st = base + k * SCH
            pltpu.sync_copy(rid.at[pl.ds(st, SCH)], idx_v)
            pltpu.sync_copy(mlat.at[pl.ds(st, SCH)], rows_v)
            pltpu.sync_copy(rows_v, acc_sh.at[idx_v], add=True)

    @pl.when(c == 1)
    def _world():
        for k in range(NCH_S):
            st = base + k * SCH
            pltpu.sync_copy(wrid.at[pl.ds(st, SCH)], idx_v)
            pltpu.sync_copy(wlat.at[pl.ds(st, SCH)], rows_v)
            pltpu.sync_copy(rows_v, acc_sh.at[idx_v], add=True)

    plsc.subcore_barrier()
    for z in range(NZB):
        rows = s * (NZB * SCH) + z * SCH

        @pl.when(c == 0)
        def _out_m():
            pltpu.sync_copy(acc_sh.at[pl.ds(rows, SCH)],
                            agg_m.at[pl.ds(rows, SCH)])

        @pl.when(c == 1)
        def _out_w():
            pltpu.sync_copy(acc_sh.at[pl.ds(rows, SCH)],
                            agg_w.at[pl.ds(rows, SCH)])


def _scatter2(mesh_lat, world_lat, rid, wrid):
    return pl.kernel(
        _scatter2_body,
        out_type=[
            jax.ShapeDtypeStruct((NAGG, L), jnp.float32),
            jax.ShapeDtypeStruct((NAGG, L), jnp.float32),
        ],
        mesh=plsc.VectorSubcoreMesh(core_axis_name="c", subcore_axis_name="s"),
        compiler_params=pltpu.CompilerParams(needs_layout_passes=False),
        scratch_types=[
            pltpu.VMEM((SCH,), jnp.int32),
            pltpu.VMEM((SCH, L), jnp.float32),
            pltpu.VMEM_SHARED((NAGG, L), jnp.float32),
        ],
    )(mesh_lat, world_lat, rid, wrid, jnp.zeros((SCH, L), jnp.float32))


def _world_edge_lists(world_pos, node_type, uniq, s0, r0):
    obstacle = node_type[:, 0] == OBSTACLE
    colmask = jnp.pad(~obstacle, (0, NPAD - N))
    packed = _radius_packed(world_pos, colmask)
    sd = jnp.concatenate([s0, r0])
    rd = jnp.concatenate([r0, s0])
    dup_self = jnp.concatenate(
        [jnp.zeros_like(s0, jnp.bool_), s0 == r0])
    ok = (sd < N) & (rd < N) & ~dup_self
    flat_idx = jnp.where(ok, (sd // 16) * NPAD + rd, NG * NPAD)
    clear = jnp.zeros((NG * NPAD,), jnp.int32).at[flat_idx].add(
        jax.lax.shift_left(jnp.int32(1), sd % 16), mode='drop')
    packed = packed & ~clear.reshape(NG, NPAD)
    counts = jnp.sum(jax.lax.population_count(packed), axis=1)
    ccl = jnp.minimum(counts, GCAP - 16)
    slots = 16 * (ccl // 16 + 1)
    off = jnp.concatenate([jnp.zeros((1,), jnp.int32),
                           jnp.cumsum(slots)[:-1].astype(jnp.int32)])
    okg = (off + slots) <= EW
    cntm = jnp.where(okg, ccl, -1).astype(jnp.int32)
    offm = jnp.where(okg, off, 0).astype(jnp.int32)
    meta = jnp.zeros((NSUB, 64), jnp.int32)
    meta = meta.at[:, 0:GPS].set(offm.reshape(NSUB, GPS))
    meta = meta.at[:, 32:32 + GPS].set(cntm.reshape(NSUB, GPS))
    ws, wr = _expand_edges(packed, meta)
    total = jnp.sum(jnp.where(okg, slots, 0))
    valid = jnp.arange(EW) < total
    ws = jnp.where(valid, ws, N)
    wr = jnp.where(valid, wr, N)
    return ws, wr


def _fused_mlp_body(nx, weighted, ln, res_idx, *refs):
    xs = refs[:nx]
    nw = sum(weighted)
    w1s = refs[nx:nx + nw]
    b1_ref, w2_ref, b2_ref = refs[nx + nw:nx + nw + 3]
    out_ref = refs[-1]
    h = b1_ref[...]
    wi = 0
    for i in range(nx):
        x = xs[i][...]
        if weighted[i]:
            h = h + jax.lax.dot_general(
                x, w1s[wi][...], (((1,), (0,)), ((), ())),
                preferred_element_type=jnp.float32)
            wi += 1
        else:
            h = h + x
    h = jnp.maximum(h, 0.0)
    o = jax.lax.dot_general(
        h, w2_ref[...], (((1,), (0,)), ((), ())),
        preferred_element_type=jnp.float32) + b2_ref[...]
    if ln:
        m = jnp.mean(o, axis=-1, keepdims=True)
        d = o - m
        v = jnp.mean(d * d, axis=-1, keepdims=True)
        o = d * jax.lax.rsqrt(v + 1e-5)
    if res_idx is not None:
        o = o + xs[res_idx][...]
    out_ref[...] = o


def _fused_mlp(xs, w1s, b1, w2, b2, ln=True, res_idx=None, dout=L):
    M = xs[0].shape[0]
    weighted = tuple(w is not None for w in w1s)
    body = functools.partial(_fused_mlp_body, len(xs), weighted, ln, res_idx)
    in_specs = []
    args = []
    for x in xs:
        d = x.shape[1]
        in_specs.append(pl.BlockSpec((MLP_BLK, d), lambda i: (i, 0)))
        args.append(x)
    for w in w1s:
        if w is not None:
            in_specs.append(pl.BlockSpec(w.shape, lambda i: (0, 0)))
            args.append(w)
    for c in (b1.reshape(1, -1), w2, b2.reshape(1, -1)):
        in_specs.append(pl.BlockSpec(c.shape, lambda i: (0, 0)))
        args.append(c)
    return pl.pallas_call(
        body,
        grid=(M // MLP_BLK,),
        in_specs=in_specs,
        out_specs=pl.BlockSpec((MLP_BLK, dout), lambda i: (i, 0)),
        out_shape=jax.ShapeDtypeStruct((M, dout), jnp.float32),
    )(*args)


def _matmul_kernel(x_ref, w_ref, out_ref):
    out_ref[...] = jax.lax.dot_general(
        x_ref[...], w_ref[...], (((1,), (0,)), ((), ())),
        preferred_element_type=jnp.float32)


def _matmul(x, w):
    M = x.shape[0]
    return pl.pallas_call(
        _matmul_kernel,
        grid=(M // MLP_BLK,),
        in_specs=[
            pl.BlockSpec((MLP_BLK, x.shape[1]), lambda i: (i, 0)),
            pl.BlockSpec(w.shape, lambda i: (0, 0)),
        ],
        out_specs=pl.BlockSpec((MLP_BLK, w.shape[1]), lambda i: (i, 0)),
        out_shape=jax.ShapeDtypeStruct((M, w.shape[1]), jnp.float32),
    )(x, w)


def _pad_rows(x, M):
    return jnp.pad(x, ((0, M - x.shape[0]), (0, 0)))


def _pad_cols(x, D):
    return jnp.pad(x, ((0, 0), (0, D - x.shape[1])))


def _safe_norm(x):
    return jnp.sqrt(jnp.sum(x * x, axis=-1, keepdims=True) + 1e-12)


def _mesh_edge_lists(cells):
    e = jnp.concatenate([cells[:, 0:2], cells[:, 1:3],
                         jnp.stack([cells[:, 2], cells[:, 0]], axis=1)], axis=0)
    lo = jnp.minimum(e[:, 0], e[:, 1])
    hi = jnp.maximum(e[:, 0], e[:, 1])
    uniq = jnp.unique(lo * N + hi, size=e.shape[0], fill_value=N * N)
    valid = uniq < N * N
    s0 = jnp.where(valid, uniq // N, N).astype(jnp.int32)
    r0 = jnp.where(valid, uniq % N, N).astype(jnp.int32)
    senders = jnp.concatenate([s0, r0])
    receivers = jnp.concatenate([r0, s0])
    return senders, receivers, uniq, s0, r0


def kernel(world_pos, prev_world_pos, mesh_pos, node_type, cells, params):
    p = params
    senders, receivers, uniq, s0, r0 = _mesh_edge_lists(cells)
    ws, wr = _world_edge_lists(world_pos, node_type, uniq, s0, r0)

    velocity = world_pos - prev_world_pos
    one_hot = jax.nn.one_hot(node_type[:, 0], NODE_TYPE_SIZE, dtype=jnp.float32)
    node_feats = _pad_rows(_pad_cols(
        jnp.concatenate([velocity, one_hot], axis=-1), 16), NPAD)
    node_lat = _fused_mlp(
        [node_feats], [_pad_rows(p['node_enc_w1'], 16)],
        p['node_enc_b1'], p['node_enc_w2'], p['node_enc_b2'])

    me_w1 = p['me_w1']
    we_w1 = p['we_w1']
    nd_w1 = p['nd_w1']
    pcat_w = jnp.concatenate(
        [me_w1[:L], me_w1[L:2 * L], we_w1[:L], we_w1[L:2 * L]], axis=1)

    EP = EM_PAD
    sid_cl = jnp.pad(jnp.minimum(senders, N - 1), (0, EP - EM))
    rid_cl = jnp.pad(jnp.minimum(receivers, N - 1), (0, EP - EM))
    rid_pad = jnp.pad(receivers, (0, EP - EM), constant_values=N)
    gidx = jnp.concatenate([
        sid_cl, rid_cl + N,
        jnp.minimum(ws, N - 1) + 2 * N, jnp.minimum(wr, N - 1) + 3 * N])
    sidx = jnp.concatenate([rid_pad, wr + (N + 1)])

    ftab = jnp.concatenate(
        [world_pos, mesh_pos, jnp.zeros((N, 3), jnp.float32)], axis=1)
    fall = jnp.tile(ftab, (4, 1))[gidx]
    relwm = fall[0:EP, 0:3] - fall[EP:2 * EP, 0:3]
    relm = fall[0:EP, 3:5] - fall[EP:2 * EP, 3:5]
    relw = fall[3 * EP:4 * EP, 0:3] - fall[2 * EP:3 * EP, 0:3]
    world_feats = _pad_cols(
        jnp.concatenate([relw, _safe_norm(relw)], axis=-1), 8)
    world_lat = _fused_mlp(
        [world_feats], [_pad_rows(p['world_enc_w1'], 8)],
        p['world_enc_b1'], p['world_enc_w2'], p['world_enc_b2'])
    mesh_feats = _pad_cols(jnp.concatenate(
        [relwm, _safe_norm(relwm), relm, _safe_norm(relm)], axis=-1), 8)
    mesh_lat = _fused_mlp(
        [mesh_feats], [_pad_rows(p['mesh_enc_w1'], 8)],
        p['mesh_enc_b1'], p['mesh_enc_w2'], p['mesh_enc_b2'])

    for _ in range(STEPS):
        pcat = _matmul(node_lat, pcat_w)[:N]
        tab4 = jnp.concatenate(
            [pcat[:, 0:L], pcat[:, L:2 * L],
             pcat[:, 2 * L:3 * L], pcat[:, 3 * L:4 * L]], axis=0)
        gall = tab4[gidx]

        mesh_lat = _fused_mlp(
            [gall[0:EP], gall[EP:2 * EP], mesh_lat],
            [None, None, me_w1[2 * L:]],
            p['me_b1'], p['me_w2'], p['me_b2'], res_idx=2)
        world_lat = _fused_mlp(
            [gall[2 * EP:3 * EP], gall[3 * EP:4 * EP], world_lat],
            [None, None, we_w1[2 * L:]],
            p['we_b1'], p['we_w2'], p['we_b2'], res_idx=2)

        agg = jax.ops.segment_sum(
            jnp.concatenate([mesh_lat, world_lat], axis=0), sidx,
            num_segments=2 * N + 1)
        node_lat = _fused_mlp(
            [node_lat, _pad_rows(agg[:N], NPAD),
             _pad_rows(agg[N + 1:2 * N + 1], NPAD)],
            [nd_w1[:L], nd_w1[L:2 * L], nd_w1[2 * L:]],
            p['nd_b1'], p['nd_w2'], p['nd_b2'], res_idx=0)

    out = _fused_mlp(
        [node_lat], [p['dec_w1']],
        p['dec_b1'], _pad_cols(p['dec_w2'], L),
        jnp.pad(p['dec_b2'], (0, L - 3)), ln=False)
    return out[:N, :3]


# speedup vs baseline: 2.3172x; 1.2474x over previous
import functools

import jax
import jax.numpy as jnp
from jax.experimental import pallas as pl
from jax.experimental.pallas import tpu as pltpu
from jax.experimental.pallas import tpu_sc as plsc

N = 10000
T = 20000
L = 128
NODE_TYPE_SIZE = 9
OBSTACLE = 1
RADIUS = 0.03
STEPS = 2
WORLD_EDGE_CAP = 131072

NPAD = 10240
EM = 6 * T
EM_PAD = 120320
EW = WORLD_EDGE_CAP
ROW_BLK = 256
MLP_BLK = 512

NG = NPAD // 16
NSUB = 32
GPS = NG // NSUB
GCAP = 4096


def _radius_packed_kernel(wp_ref, wpt_ref, x2_ref, colmask_ref, packed_ref):
    i = pl.program_id(0)
    wp = wp_ref[...]
    row_sq = jnp.sum(wp * wp, axis=1, keepdims=True)
    cross = jax.lax.dot_general(
        wp, wpt_ref[...], (((1,), (0,)), ((), ())),
        preferred_element_type=jnp.float32)
    d2 = row_sq + x2_ref[...] - 2.0 * cross
    dist = jnp.sqrt(jnp.maximum(d2, 0.0))
    rows = i * ROW_BLK + jax.lax.broadcasted_iota(jnp.int32, (ROW_BLK, NPAD), 0)
    cols = jax.lax.broadcasted_iota(jnp.int32, (ROW_BLK, NPAD), 1)
    conn = (dist < RADIUS) & (rows != cols) & (rows < N) & (cols < N)
    conn = conn & colmask_ref[...]
    rr = jax.lax.broadcasted_iota(jnp.int32, (16, ROW_BLK), 1)
    tt = jax.lax.broadcasted_iota(jnp.int32, (16, ROW_BLK), 0)
    a = jnp.where(rr // 16 == tt,
                  jax.lax.shift_left(jnp.int32(1), rr % 16), 0
                  ).astype(jnp.float32)
    packed_f = jax.lax.dot_general(
        a, conn.astype(jnp.float32), (((1,), (0,)), ((), ())),
        preferred_element_type=jnp.float32)
    packed_ref[...] = packed_f.astype(jnp.int32)


def _radius_packed(world_pos, colmask):
    wp_pad = jnp.zeros((NPAD, 8), jnp.float32)
    wp_pad = wp_pad.at[:, 0].set(1e6)
    wp_pad = wp_pad.at[:N, :3].set(world_pos)
    wp_pad = wp_pad.at[:N, 3:].set(0.0)
    x2 = jnp.sum(wp_pad * wp_pad, axis=1)[None, :]
    packed = pl.pallas_call(
        _radius_packed_kernel,
        grid=(NPAD // ROW_BLK,),
        in_specs=[
            pl.BlockSpec((ROW_BLK, 8), lambda i: (i, 0)),
            pl.BlockSpec((8, NPAD), lambda i: (0, 0)),
            pl.BlockSpec((1, NPAD), lambda i: (0, 0)),
            pl.BlockSpec((1, NPAD), lambda i: (0, 0)),
        ],
        out_specs=pl.BlockSpec((16, NPAD), lambda i: (i, 0)),
        out_shape=jax.ShapeDtypeStruct((NG, NPAD), jnp.int32),
    )(wp_pad, wp_pad.T, x2, colmask.reshape(1, NPAD))
    return packed


def _sget(vec, chunk, lane):
    v = vec[pl.ds(chunk * 16, 16)]
    return jnp.sum(jnp.where(jax.lax.iota(jnp.int32, 16) == lane, v, 0))


def _expand_body(packed_hbm, meta_hbm, ws_hbm, wr_hbm,
                 meta_v, row_v, stage_s, stage_r):
    c = jax.lax.axis_index("c")
    s = jax.lax.axis_index("s")
    wid = s * 2 + c
    pltpu.sync_copy(meta_hbm.at[wid], meta_v)
    for g in range(GPS):
        off = _sget(meta_v, g // 16, g % 16)
        cnt = _sget(meta_v, 2 + g // 16, g % 16)
        grp = wid * GPS + g

        @pl.when(cnt >= 0)
        def _process():
            pltpu.sync_copy(packed_hbm.at[grp], row_v)

            def scan_body(k, fill):
                w = row_v[pl.ds(k * 16, 16)]
                nz = jnp.sum(jnp.where(w != 0, 1, 0))

                def emit(f):
                    for b in range(16):
                        maskb = ((w >> b) & 1) == 1
                        cb = jnp.sum(jnp.where(maskb, 1, 0))

                        def do(f2):
                            f2c = jnp.minimum(f2, GCAP - 16)
                            cols = k * 16 + jax.lax.iota(jnp.int32, 16)
                            rowv = jnp.full((16,), 0, jnp.int32) + grp * 16 + b
                            plsc.store_compressed(
                                stage_s.at[pl.ds(f2c, 16)], rowv, mask=maskb)
                            plsc.store_compressed(
                                stage_r.at[pl.ds(f2c, 16)], cols, mask=maskb)
                            return f2 + cb

                        f = jax.lax.cond(cb > 0, do, lambda f2: f2, f)
                    return f

                return jax.lax.cond(nz > 0, emit, lambda f: f, fill)

            fill = jax.lax.fori_loop(0, NPAD // 16, scan_body, 0)
            fillc = jnp.minimum(fill, GCAP - 16)
            sent = jnp.full((16,), 0, jnp.int32) + N
            stage_s[pl.ds(fillc, 16)] = sent
            stage_r[pl.ds(fillc, 16)] = sent

            def drain(ci, _):
                src = pl.multiple_of(ci * 16, 16)
                dst = pl.multiple_of(off + ci * 16, 16)
                pltpu.sync_copy(stage_s.at[pl.ds(src, 16)],
                                ws_hbm.at[pl.ds(dst, 16)])
                pltpu.sync_copy(stage_r.at[pl.ds(src, 16)],
                                wr_hbm.at[pl.ds(dst, 16)])
                return 0

            jax.lax.fori_loop(0, fillc // 16 + 1, drain, 0)


def _expand_edges(packed, meta):
    return pl.kernel(
        _expand_body,
        out_type=[
            jax.ShapeDtypeStruct((EW,), jnp.int32),
            jax.ShapeDtypeStruct((EW,), jnp.int32),
        ],
        mesh=plsc.VectorSubcoreMesh(core_axis_name="c", subcore_axis_name="s"),
        compiler_params=pltpu.CompilerParams(needs_layout_passes=False),
        scratch_types=[
            pltpu.VMEM((64,), jnp.int32),
            pltpu.VMEM((NPAD,), jnp.int32),
            pltpu.VMEM((GCAP + 16,), jnp.int32),
            pltpu.VMEM((GCAP + 16,), jnp.int32),
        ],
    )(packed, meta)


def _world_edge_lists(world_pos, node_type, uniq, s0, r0):
    obstacle = node_type[:, 0] == OBSTACLE
    colmask = jnp.pad(~obstacle, (0, NPAD - N))
    packed = _radius_packed(world_pos, colmask)
    sd = jnp.concatenate([s0, r0])
    rd = jnp.concatenate([r0, s0])
    dup_self = jnp.concatenate(
        [jnp.zeros_like(s0, jnp.bool_), s0 == r0])
    ok = (sd < N) & (rd < N) & ~dup_self
    flat_idx = jnp.where(ok, (sd // 16) * NPAD + rd, NG * NPAD)
    clear = jnp.zeros((NG * NPAD,), jnp.int32).at[flat_idx].add(
        jax.lax.shift_left(jnp.int32(1), sd % 16), mode='drop')
    packed = packed & ~clear.reshape(NG, NPAD)
    counts = jnp.sum(jax.lax.population_count(packed), axis=1)
    ccl = jnp.minimum(counts, GCAP - 16)
    slots = 16 * (ccl // 16 + 1)
    off = jnp.concatenate([jnp.zeros((1,), jnp.int32),
                           jnp.cumsum(slots)[:-1].astype(jnp.int32)])
    okg = (off + slots) <= EW
    cntm = jnp.where(okg, ccl, -1).astype(jnp.int32)
    offm = jnp.where(okg, off, 0).astype(jnp.int32)
    meta = jnp.zeros((NSUB, 64), jnp.int32)
    meta = meta.at[:, 0:GPS].set(offm.reshape(NSUB, GPS))
    meta = meta.at[:, 32:32 + GPS].set(cntm.reshape(NSUB, GPS))
    ws, wr = _expand_edges(packed, meta)
    total = jnp.sum(jnp.where(okg, slots, 0))
    valid = jnp.arange(EW) < total
    ws = jnp.where(valid, ws, N)
    wr = jnp.where(valid, wr, N)
    return ws, wr


def _fused_mlp_body(nx, weighted, ln, res_idx, *refs):
    xs = refs[:nx]
    nw = sum(weighted)
    w1s = refs[nx:nx + nw]
    b1_ref, w2_ref, b2_ref = refs[nx + nw:nx + nw + 3]
    out_ref = refs[-1]
    h = b1_ref[...]
    wi = 0
    for i in range(nx):
        x = xs[i][...]
        if weighted[i]:
            h = h + jax.lax.dot_general(
                x, w1s[wi][...], (((1,), (0,)), ((), ())),
                preferred_element_type=jnp.float32)
            wi += 1
        else:
            h = h + x
    h = jnp.maximum(h, 0.0)
    o = jax.lax.dot_general(
        h, w2_ref[...], (((1,), (0,)), ((), ())),
        preferred_element_type=jnp.float32) + b2_ref[...]
    if ln:
        m = jnp.mean(o, axis=-1, keepdims=True)
        d = o - m
        v = jnp.mean(d * d, axis=-1, keepdims=True)
        o = d * jax.lax.rsqrt(v + 1e-5)
    if res_idx is not None:
        o = o + xs[res_idx][...]
    out_ref[...] = o


def _fused_mlp(xs, w1s, b1, w2, b2, ln=True, res_idx=None, dout=L):
    M = xs[0].shape[0]
    weighted = tuple(w is not None for w in w1s)
    body = functools.partial(_fused_mlp_body, len(xs), weighted, ln, res_idx)
    in_specs = []
    args = []
    for x in xs:
        d = x.shape[1]
        in_specs.append(pl.BlockSpec((MLP_BLK, d), lambda i: (i, 0)))
        args.append(x)
    for w in w1s:
        if w is not None:
            in_specs.append(pl.BlockSpec(w.shape, lambda i: (0, 0)))
            args.append(w)
    for c in (b1.reshape(1, -1), w2, b2.reshape(1, -1)):
        in_specs.append(pl.BlockSpec(c.shape, lambda i: (0, 0)))
        args.append(c)
    return pl.pallas_call(
        body,
        grid=(M // MLP_BLK,),
        in_specs=in_specs,
        out_specs=pl.BlockSpec((MLP_BLK, dout), lambda i: (i, 0)),
        out_shape=jax.ShapeDtypeStruct((M, dout), jnp.float32),
    )(*args)


def _matmul_kernel(x_ref, w_ref, out_ref):
    out_ref[...] = jax.lax.dot_general(
        x_ref[...], w_ref[...], (((1,), (0,)), ((), ())),
        preferred_element_type=jnp.float32)


def _matmul(x, w):
    M = x.shape[0]
    return pl.pallas_call(
        _matmul_kernel,
        grid=(M // MLP_BLK,),
        in_specs=[
            pl.BlockSpec((MLP_BLK, x.shape[1]), lambda i: (i, 0)),
            pl.BlockSpec(w.shape, lambda i: (0, 0)),
        ],
        out_specs=pl.BlockSpec((MLP_BLK, w.shape[1]), lambda i: (i, 0)),
        out_shape=jax.ShapeDtypeStruct((M, w.shape[1]), jnp.float32),
    )(x, w)


def _pad_rows(x, M):
    return jnp.pad(x, ((0, M - x.shape[0]), (0, 0)))


def _pad_cols(x, D):
    return jnp.pad(x, ((0, 0), (0, D - x.shape[1])))


def _safe_norm(x):
    return jnp.sqrt(jnp.sum(x * x, axis=-1, keepdims=True) + 1e-12)


def _mesh_edge_lists(cells):
    e = jnp.concatenate([cells[:, 0:2], cells[:, 1:3],
                         jnp.stack([cells[:, 2], cells[:, 0]], axis=1)], axis=0)
    lo = jnp.minimum(e[:, 0], e[:, 1])
    hi = jnp.maximum(e[:, 0], e[:, 1])
    uniq = jnp.unique(lo * N + hi, size=e.shape[0], fill_value=N * N)
    valid = uniq < N * N
    s0 = jnp.where(valid, uniq // N, N).astype(jnp.int32)
    r0 = jnp.where(valid, uniq % N, N).astype(jnp.int32)
    senders = jnp.concatenate([s0, r0])
    receivers = jnp.concatenate([r0, s0])
    return senders, receivers, uniq, s0, r0


def kernel(world_pos, prev_world_pos, mesh_pos, node_type, cells, params):
    p = params
    senders, receivers, uniq, s0, r0 = _mesh_edge_lists(cells)
    ws, wr = _world_edge_lists(world_pos, node_type, uniq, s0, r0)

    velocity = world_pos - prev_world_pos
    one_hot = jax.nn.one_hot(node_type[:, 0], NODE_TYPE_SIZE, dtype=jnp.float32)
    node_feats = _pad_rows(_pad_cols(
        jnp.concatenate([velocity, one_hot], axis=-1), 16), NPAD)
    node_lat = _fused_mlp(
        [node_feats], [_pad_rows(p['node_enc_w1'], 16)],
        p['node_enc_b1'], p['node_enc_w2'], p['node_enc_b2'])

    relw = world_pos[wr] - world_pos[ws]
    world_feats = _pad_cols(
        jnp.concatenate([relw, _safe_norm(relw)], axis=-1), 8)
    world_lat = _fused_mlp(
        [world_feats], [_pad_rows(p['world_enc_w1'], 8)],
        p['world_enc_b1'], p['world_enc_w2'], p['world_enc_b2'])

    relwm = world_pos[senders] - world_pos[receivers]
    relm = mesh_pos[senders] - mesh_pos[receivers]
    mesh_feats = _pad_rows(_pad_cols(jnp.concatenate(
        [relwm, _safe_norm(relwm), relm, _safe_norm(relm)], axis=-1), 8), EM_PAD)
    mesh_lat = _fused_mlp(
        [mesh_feats], [_pad_rows(p['mesh_enc_w1'], 8)],
        p['mesh_enc_b1'], p['mesh_enc_w2'], p['mesh_enc_b2'])

    me_w1 = p['me_w1']
    we_w1 = p['we_w1']
    nd_w1 = p['nd_w1']
    pcat_w = jnp.concatenate(
        [me_w1[:L], me_w1[L:2 * L], we_w1[:L], we_w1[L:2 * L]], axis=1)

    sid_pad = jnp.pad(senders, (0, EM_PAD - EM), constant_values=N)
    rid_pad = jnp.pad(receivers, (0, EM_PAD - EM), constant_values=N)

    for _ in range(STEPS):
        pcat = _matmul(node_lat, pcat_w)[:N]
        ps_me = pcat[:, 0:L]
        pr_me = pcat[:, L:2 * L]
        ps_we = pcat[:, 2 * L:3 * L]
        pr_we = pcat[:, 3 * L:4 * L]
        g_me = ps_me[sid_pad] + pr_me[rid_pad]
        g_we = ps_we[ws] + pr_we[wr]

        mesh_lat = _fused_mlp(
            [g_me, mesh_lat], [None, me_w1[2 * L:]],
            p['me_b1'], p['me_w2'], p['me_b2'], res_idx=1)
        world_lat = _fused_mlp(
            [g_we, world_lat], [None, we_w1[2 * L:]],
            p['we_b1'], p['we_w2'], p['we_b2'], res_idx=1)

        agg_m = jax.ops.segment_sum(mesh_lat, rid_pad, num_segments=N)
        agg_w = jax.ops.segment_sum(world_lat, wr, num_segments=N)
        node_lat = _fused_mlp(
            [node_lat, _pad_rows(agg_m, NPAD), _pad_rows(agg_w, NPAD)],
            [nd_w1[:L], nd_w1[L:2 * L], nd_w1[2 * L:]],
            p['nd_b1'], p['nd_w2'], p['nd_b2'], res_idx=0)

    out = _fused_mlp(
        [node_lat], [p['dec_w1']],
        p['dec_b1'], _pad_cols(p['dec_w2'], L),
        jnp.pad(p['dec_b2'], (0, L - 3)), ln=False)
    return out[:N, :3]
